# Initial kernel scaffold; baseline (speedup 1.0000x reference)
import jax
import jax.numpy as jnp
from jax.experimental import pallas as pl


def kernel(out0, out1, out2, out3, conv0_w, conv0_b, conv1_w, conv1_b, conv2_w, conv2_b, conv3_w, conv3_b, x_proj_weight, dt_projs_weight, dt_projs_bias, A_logs, Ds, ln_g, ln_b):
    raise NotImplementedError("write your pallas kernel here")


# speedup vs baseline: 58.0359x; 58.0359x over previous
import functools

import numpy as np
import jax
import jax.numpy as jnp
from jax.experimental import pallas as pl
from jax.experimental.pallas import tpu as pltpu

D_IN = 32
N_ST = 16
R_DT = 32
DN = D_IN * N_ST


def _cubic_matrix_np(out_size, in_size):
    a = -0.75
    src = np.arange(out_size, dtype=np.float64) * (float(in_size - 1) / float(out_size - 1))
    i0 = np.floor(src).astype(np.int64)
    idx = i0[:, None] + np.arange(-1, 3, dtype=np.int64)[None, :]
    d = np.abs(src[:, None] - idx.astype(np.float64))
    w = np.where(d <= 1.0, (a + 2.0) * d**3 - (a + 3.0) * d**2 + 1.0,
                 np.where(d < 2.0, a * d**3 - 5.0 * a * d**2 + 8.0 * a * d - 4.0 * a, 0.0))
    idxc = np.clip(idx, 0, in_size - 1)
    M = np.zeros((out_size, in_size), np.float64)
    np.add.at(M, (np.arange(out_size)[:, None].repeat(4, 1), idxc), w)
    return M.astype(np.float32)


def _expand_d_np():
    E = np.zeros((D_IN, DN), np.float32)
    for n in range(N_ST):
        for d in range(D_IN):
            E[d, n * D_IN + d] = 1.0
    return E


def _expand_n_np():
    E = np.zeros((N_ST, DN), np.float32)
    for n in range(N_ST):
        for d in range(D_IN):
            E[n, n * D_IN + d] = 1.0
    return E


def _reduce_n_np():
    R = np.zeros((DN, D_IN), np.float32)
    for n in range(N_ST):
        for d in range(D_IN):
            R[n * D_IN + d, d] = 1.0
    return R


def _softplus(x):
    return jnp.where(x > 20.0, x, jnp.log1p(jnp.exp(jnp.minimum(x, 20.0))))


def _dot(a, b):
    return jax.lax.dot_general(a, b, (((1,), (0,)), ((), ())),
                               preferred_element_type=jnp.float32)


def _scan_chunk(ch, h, A_fl, ds_k, ed_ref, en_ref, rn_ref, T):
    d_c = ch[:, 0:32]
    u_c = ch[:, 32:64]
    B_c = ch[:, 64:80]
    C_c = ch[:, 80:96]
    d_rep = _dot(d_c, ed_ref[:])
    du_rep = _dot(d_c * u_c, ed_ref[:])
    B_rep = _dot(B_c, en_ref[:])
    C_rep = _dot(C_c, en_ref[:])
    a = jnp.exp(d_rep * A_fl)
    b = du_rep * B_rep
    s = 1
    while s < T:
        ia = jnp.ones((s, DN), jnp.float32)
        ib = jnp.zeros((s, DN), jnp.float32)
        a_sh = jnp.concatenate([ia, a[:T - s]], axis=0)
        b_sh = jnp.concatenate([ib, b[:T - s]], axis=0)
        b = a * b_sh + b
        a = a * a_sh
        s *= 2
    hs = b + a * h
    y = _dot(C_rep * hs, rn_ref[:]) + u_c * ds_k
    return y, hs[T - 1:T, :]


def _conv0_body(x_ref, w_ref, b_ref, o_ref, *, H, W, C):
    acc = None
    xp = x_ref[0]
    for t in range(9):
        dy, dx = t // 3, t % 3
        p = xp[dy:dy + H, dx:dx + W, :].reshape(H * W, C)
        z = _dot(p, w_ref[t])
        acc = z if acc is None else acc + z
    o_ref[0] = acc + b_ref[:]


def _stage_body(fus_ref, featp_ref, wup_ref, wf_ref, cb_ref, mh_ref,
                mwt_ref, xproj_ref, dtw_ref, dtb_ref, alog_ref, ds_ref,
                lng_ref, lnb_ref, ed_ref, en_ref, rn_ref, o_ref,
                u1_sc, uimg_sc, in0_sc, in1_sc, out_sc,
                *, H, W, h, w, C_feat):
    L = H * W
    nB = 4 if H >= 64 else 2
    Hb = H // nB

    U1 = _dot(mh_ref[:], fus_ref[0])
    for ub in range(nB):
        r0 = ub * Hb
        u1_sc[:] = U1[r0:r0 + Hb].reshape(Hb, w, D_IN)
        t1 = jnp.swapaxes(u1_sc[:], 1, 2)
        U2 = _dot(t1.reshape(Hb * D_IN, w), mwt_ref[:])
        uimg_sc[1 + r0:1 + r0 + Hb, 1:W + 1, :] = (
            jnp.swapaxes(U2.reshape(Hb, D_IN, W), 1, 2))

    zrow = jnp.zeros((1, W + 2, D_IN), jnp.float32)
    uimg_sc[0:1] = zrow
    uimg_sc[H + 1:H + 2] = zrow
    zcol = jnp.zeros((H + 2, 1, D_IN), jnp.float32)
    uimg_sc[:, 0:1, :] = zcol
    uimg_sc[:, W + 1:W + 2, :] = zcol

    for rb in range(nB):
        r0 = rb * Hb
        acc = None
        for t in range(9):
            dy, dx = t // 3, t % 3
            pu = uimg_sc[r0 + dy:r0 + dy + Hb, dx:dx + W, :].reshape(Hb * W, D_IN)
            pf = featp_ref[0, r0 + dy:r0 + dy + Hb, dx:dx + W, :].reshape(Hb * W, C_feat)
            z = _dot(pu, wup_ref[t]) + _dot(pf, wf_ref[t])
            acc = z if acc is None else acc + z
        x_b = acc + cb_ref[:]
        for k, sc in ((0, in0_sc), (1, in1_sc)):
            xd = _dot(x_b, xproj_ref[k])
            dts = _dot(xd[:, 0:R_DT], dtw_ref[k])
            delta = _softplus(dts + dtb_ref[k])
            sc[r0:r0 + Hb, :, 0:32] = delta.reshape(Hb, W, D_IN)
            sc[r0:r0 + Hb, :, 32:64] = x_b.reshape(Hb, W, D_IN)
            sc[r0:r0 + Hb, :, 64:80] = xd[:, 32:48].reshape(Hb, W, N_ST)
            sc[r0:r0 + Hb, :, 80:96] = xd[:, 48:64].reshape(Hb, W, N_ST)

    h0 = jnp.zeros((1, DN), jnp.float32)
    A0 = -jnp.exp(alog_ref[0])
    A1 = -jnp.exp(alog_ref[1])

    def body0(c, hc):
        y, hn = _scan_chunk(in0_sc[c], hc, A0, ds_ref[0],
                            ed_ref, en_ref, rn_ref, W)
        in0_sc[c, :, 96:128] = y
        return hn
    jax.lax.fori_loop(0, H, body0, h0)

    def body1(j, hc):
        ch = in1_sc[:, pl.ds(j, 1), :].reshape(H, 128)
        y, hn = _scan_chunk(ch, hc, A1, ds_ref[1],
                            ed_ref, en_ref, rn_ref, H)
        in1_sc[:, pl.ds(j, 1), 96:128] = y.reshape(H, 1, D_IN)
        return hn
    jax.lax.fori_loop(0, W, body1, h0)

    for rb in range(nB):
        r0 = rb * Hb
        ysum = in0_sc[r0:r0 + Hb, :, 96:128] + in1_sc[r0:r0 + Hb, :, 96:128]
        mu = jnp.mean(ysum, axis=2, keepdims=True)
        var = jnp.mean((ysum - mu) ** 2, axis=2, keepdims=True)
        out = (ysum - mu) * jax.lax.rsqrt(var + 1e-5)
        out = out * lng_ref[:].reshape(1, 1, D_IN) + lnb_ref[:].reshape(1, 1, D_IN)
        out_sc[r0:r0 + Hb] = out.reshape(Hb, W * D_IN)
    o_ref[0] = out_sc[:]


def _full_spec(shape):
    return pl.BlockSpec(shape, lambda b: (0,) * len(shape))


def _run_conv0(xp, w_taps, bias):
    Bsz, Hp, Wp, C = xp.shape
    H, W = Hp - 2, Wp - 2
    return pl.pallas_call(
        functools.partial(_conv0_body, H=H, W=W, C=C),
        grid=(Bsz,),
        in_specs=[
            pl.BlockSpec((1, Hp, Wp, C), lambda b: (b, 0, 0, 0)),
            _full_spec(w_taps.shape),
            _full_spec(bias.shape),
        ],
        out_specs=pl.BlockSpec((1, H * W, D_IN), lambda b: (b, 0, 0)),
        out_shape=jax.ShapeDtypeStruct((Bsz, H * W, D_IN), jnp.float32),
        compiler_params=pltpu.CompilerParams(
            dimension_semantics=("parallel",)),
    )(xp, w_taps, bias).reshape(Bsz, H, W * D_IN)


def _run_stage(fusion, featp, wup, wf, cb, xprojT, dtwT, dtb, alogT,
               ds2, lng, lnb, ed, en, rn):
    Bsz, h, w32 = fusion.shape
    w = w32 // D_IN
    _, Hp, Wp, C_feat = featp.shape
    H, W = Hp - 2, Wp - 2
    mh = jnp.asarray(_cubic_matrix_np(H, h))
    mwt = jnp.asarray(_cubic_matrix_np(W, w).T)
    return pl.pallas_call(
        functools.partial(_stage_body, H=H, W=W, h=h, w=w, C_feat=C_feat),
        grid=(Bsz,),
        in_specs=[
            pl.BlockSpec((1, h, w32), lambda b: (b, 0, 0)),
            pl.BlockSpec((1, Hp, Wp, C_feat), lambda b: (b, 0, 0, 0)),
            _full_spec(wup.shape),
            _full_spec(wf.shape),
            _full_spec(cb.shape),
            _full_spec(mh.shape),
            _full_spec(mwt.shape),
            _full_spec(xprojT.shape),
            _full_spec(dtwT.shape),
            _full_spec(dtb.shape),
            _full_spec(alogT.shape),
            _full_spec(ds2.shape),
            _full_spec(lng.shape),
            _full_spec(lnb.shape),
            _full_spec(ed.shape),
            _full_spec(en.shape),
            _full_spec(rn.shape),
        ],
        out_specs=pl.BlockSpec((1, H, W * D_IN), lambda b: (b, 0, 0)),
        out_shape=jax.ShapeDtypeStruct((Bsz, H, W * D_IN), jnp.float32),
        scratch_shapes=[
            pltpu.VMEM((H // (4 if H >= 64 else 2), w, D_IN), jnp.float32),
            pltpu.VMEM((H + 2, W + 2, D_IN), jnp.float32),
            pltpu.VMEM((H, W, 128), jnp.float32),
            pltpu.VMEM((H, W, 128), jnp.float32),
            pltpu.VMEM((H, W * D_IN), jnp.float32),
        ],
        compiler_params=pltpu.CompilerParams(
            dimension_semantics=("parallel",)),
    )(fusion, featp, wup, wf, cb, mh, mwt, xprojT, dtwT, dtb, alogT,
      ds2, lng, lnb, ed, en, rn)


def _taps_T(wm):
    return jnp.transpose(wm, (2, 3, 1, 0)).reshape(9, wm.shape[1], wm.shape[0])


def _pad_chw_to_hwc(x):
    xt = jnp.transpose(x, (0, 2, 3, 1))
    return jnp.pad(xt, ((0, 0), (1, 1), (1, 1), (0, 0)))


def kernel(out0, out1, out2, out3, conv0_w, conv0_b, conv1_w, conv1_b,
           conv2_w, conv2_b, conv3_w, conv3_b, x_proj_weight,
           dt_projs_weight, dt_projs_bias, A_logs, Ds, ln_g, ln_b):
    ed = jnp.asarray(_expand_d_np())
    en = jnp.asarray(_expand_n_np())
    rn = jnp.asarray(_reduce_n_np())
    alogT = A_logs.reshape(2, D_IN, N_ST).transpose(0, 2, 1).reshape(2, 1, DN)
    ds2 = Ds.reshape(2, 1, D_IN)
    dtb = dt_projs_bias.reshape(2, 1, D_IN)
    xprojT = jnp.transpose(x_proj_weight, (0, 2, 1))
    dtwT = jnp.transpose(dt_projs_weight, (0, 2, 1))
    lng = ln_g.reshape(1, D_IN)
    lnb = ln_b.reshape(1, D_IN)

    fusion = _run_conv0(_pad_chw_to_hwc(out3), _taps_T(conv0_w),
                        conv0_b.reshape(1, D_IN))
    for feat, cw, cb in ((out2, conv1_w, conv1_b),
                         (out1, conv2_w, conv2_b),
                         (out0, conv3_w, conv3_b)):
        wt = _taps_T(cw)
        fusion = _run_stage(fusion, _pad_chw_to_hwc(feat),
                            wt[:, :D_IN, :], wt[:, D_IN:, :],
                            cb.reshape(1, D_IN), xprojT, dtwT, dtb,
                            alogT, ds2, lng, lnb, ed, en, rn)
    Bsz = out0.shape[0]
    out = fusion.reshape(Bsz, 128, 128, D_IN)
    return jnp.transpose(out, (0, 3, 1, 2))
